# baseline (speedup 1.0000x reference)
import functools

import jax
import jax.numpy as jnp
from jax import lax
from jax.experimental import pallas as pl
from jax.experimental.pallas import tpu as pltpu
from jax.experimental.pallas import tpu_sc as plsc

N_FACTORS = 64
BATCH = 16384
N_USERS = 1000000
N_BOOKS = 100000

_info = plsc.get_sparse_core_info()
NC = _info.num_cores
NS = _info.num_subcores
LANES = _info.num_lanes
NW = NC * NS
BPW = BATCH // NW
GCHUNK = 128
CH = 256
NPASS = BPW // CH


def _body(user_hbm, book_hbm, ut_hbm, bt_hbm, out_hbm,
          uidx_v, bidx_v, uflat_v, bflat_v, ubuf_v, bbuf_v, out_v,
          sem_u, sem_b):
  wid = lax.axis_index("s") * NC + lax.axis_index("c")
  base = wid * BPW

  pltpu.sync_copy(user_hbm.at[pl.ds(base, BPW)], uidx_v)
  pltpu.sync_copy(book_hbm.at[pl.ds(base, BPW)], bidx_v)

  for p in range(NPASS):
    def jbody(j, carry):
      ju = j * N_USERS
      jb = j * N_BOOKS
      for g in range(CH // LANES):
        sl = pl.ds(g * LANES, LANES)
        src = pl.ds(p * CH + g * LANES, LANES)
        uflat_v[j, sl] = uidx_v[src] + ju
        bflat_v[j, sl] = bidx_v[src] + jb
      copies = []
      for c in range(CH // GCHUNK):
        csl = pl.ds(c * GCHUNK, GCHUNK)
        copies.append(pltpu.async_copy(
            ut_hbm.at[uflat_v.at[j, csl]], ubuf_v.at[j, csl], sem_u))
        copies.append(pltpu.async_copy(
            bt_hbm.at[bflat_v.at[j, csl]], bbuf_v.at[j, csl], sem_b))
      for cp in copies:
        cp.wait()
      return carry

    lax.fori_loop(0, N_FACTORS, jbody, 0)

    def gbody(g, carry):
      sl = pl.ds(g * LANES, LANES)
      acc = jnp.zeros((LANES,), jnp.float32)
      for j in range(N_FACTORS):
        acc = acc + ubuf_v[j, sl] * bbuf_v[j, sl]
      out_v[sl] = acc
      return carry

    lax.fori_loop(0, CH // LANES, gbody, 0)
    pltpu.sync_copy(out_v, out_hbm.at[pl.ds(base + p * CH, CH)])


@jax.jit
def kernel(user, book, user_emb, book_emb):
  ut = user_emb.T.reshape(-1)
  bt = book_emb.T.reshape(-1)
  mesh = plsc.VectorSubcoreMesh(core_axis_name="c", subcore_axis_name="s")
  run = functools.partial(
      pl.kernel,
      out_type=jax.ShapeDtypeStruct((BATCH,), jnp.float32),
      mesh=mesh,
      compiler_params=pltpu.CompilerParams(needs_layout_passes=False),
      scratch_types=[
          pltpu.VMEM((BPW,), jnp.int32),
          pltpu.VMEM((BPW,), jnp.int32),
          pltpu.VMEM((N_FACTORS, CH), jnp.int32),
          pltpu.VMEM((N_FACTORS, CH), jnp.int32),
          pltpu.VMEM((N_FACTORS, CH), jnp.float32),
          pltpu.VMEM((N_FACTORS, CH), jnp.float32),
          pltpu.VMEM((CH,), jnp.float32),
          pltpu.SemaphoreType.DMA,
          pltpu.SemaphoreType.DMA,
      ],
  )(_body)
  return run(user.astype(jnp.int32), book.astype(jnp.int32), ut, bt)
